# Initial kernel scaffold; baseline (speedup 1.0000x reference)
import jax
import jax.numpy as jnp
from jax.experimental import pallas as pl


def kernel(exercise_id, weight_id, exercise_sequence, equipment_id, core, metric_type, exercise_table, weight_table, seq_table, equipment_table, core_table, metric_table, weight_fc_w, weight_fc_b, seq_fc_w, seq_fc_b, equipment_fc_w, equipment_fc_b, core_fc_w, core_fc_b, metric_fc_w, metric_fc_b):
    raise NotImplementedError("write your pallas kernel here")


# speedup vs baseline: 8.8686x; 8.8686x over previous
import functools

import jax
import jax.numpy as jnp
from jax import lax
from jax.experimental import pallas as pl
from jax.experimental.pallas import tpu as pltpu
from jax.experimental.pallas import tpu_sc as plsc

D = 8
DR = 128
DO = 16
NC, NS = 2, 16
NW = NC * NS
SUB = 128
CHUNK = 128


def _prep_body(wt, st, et, ct, mt, wfw, wfb, sfw, sfb, efw, efb, cfw, cfb,
               mfw, mfb, wp_ref, ep_ref, scm_ref):
    dn = (((1,), (1,)), ((), ()))
    hi = lax.Precision.HIGHEST

    def wide(x):
        return jnp.pad(x, ((0, 0), (0, DR - D)))

    wp_ref[...] = wide(lax.dot_general(wt[...], wfw[...], dn, precision=hi,
                                       preferred_element_type=jnp.float32)
                       + wfb[...][None, :])
    ep_ref[...] = wide(lax.dot_general(et[...], efw[...][:, 0:2], dn,
                                       precision=hi,
                                       preferred_element_type=jnp.float32)
                       + efb[...][None, :])
    sp = lax.dot_general(st[...], sfw[...], dn, precision=hi,
                         preferred_element_type=jnp.float32)
    cp = lax.dot_general(ct[...], cfw[...], dn, precision=hi,
                         preferred_element_type=jnp.float32)
    mp = lax.dot_general(mt[...], mfw[...], dn, precision=hi,
                         preferred_element_type=jnp.float32)
    b3 = sfb[...] + cfb[...] + mfb[...]
    for c in range(2):
        for m in range(4):
            scm_ref[c * 4 + m, :, :] = wide(sp + (cp[c] + mp[m] + b3)[None, :])


def _sc_body(n_chunks, exi_h, wi_h, eqi_h, si_h, ci_h, mi_h, ex_t, wp_t,
             ep_t, scm_t, out_h, exi, wi, eqi, si, ci, mi, smi, g0, g1, g2,
             g3, outs, sem):
    cid = lax.axis_index("c")
    sid = lax.axis_index("s")
    wid = sid * NC + cid
    tok0 = wid * (n_chunks * CHUNK)

    def chunk_body(c, carry):
        t = tok0 + c * CHUNK
        pltpu.sync_copy(exi_h.at[pl.ds(t, CHUNK)], exi)
        pltpu.sync_copy(wi_h.at[pl.ds(t, CHUNK)], wi)
        pltpu.sync_copy(eqi_h.at[pl.ds(t, CHUNK)], eqi)
        pltpu.sync_copy(si_h.at[pl.ds(t, CHUNK)], si)
        pltpu.sync_copy(ci_h.at[pl.ds(t, CHUNK)], ci)
        pltpu.sync_copy(mi_h.at[pl.ds(t, CHUNK)], mi)
        for v in range(SUB // 16):
            sl = pl.ds(v * 16, 16)
            smi[sl] = ci[sl] * 800 + mi[sl] * 200 + si[sl]
        cps = [pltpu.async_copy(ex_t.at[exi], g0, sem),
               pltpu.async_copy(wp_t.at[wi], g1, sem),
               pltpu.async_copy(ep_t.at[eqi], g2, sem),
               pltpu.async_copy(scm_t.at[smi], g3, sem)]
        for cp in cps:
            cp.wait()
        for i in range(CHUNK):
            sl = pl.ds(0, 16)
            outs[i, sl] = ((g0[i, sl] + g1[i, sl]) + (g2[i, sl] + g3[i, sl]))
        pltpu.sync_copy(outs, out_h.at[pl.ds(t, CHUNK)])
        return carry

    lax.fori_loop(0, n_chunks, chunk_body, 0)


def kernel(exercise_id, weight_id, exercise_sequence, equipment_id, core,
           metric_type, exercise_table, weight_table, seq_table,
           equipment_table, core_table, metric_table, weight_fc_w,
           weight_fc_b, seq_fc_w, seq_fc_b, equipment_fc_w, equipment_fc_b,
           core_fc_w, core_fc_b, metric_fc_w, metric_fc_b):
    B, L = exercise_id.shape
    N = B * L
    assert N % (NW * CHUNK) == 0
    n_chunks = N // (NW * CHUNK)
    vw = weight_table.shape[0]
    vq = equipment_table.shape[0]
    vs, vc, vm = seq_table.shape[0], core_table.shape[0], metric_table.shape[0]

    wp, ep, scm3 = pl.pallas_call(
        _prep_body,
        out_shape=[
            jax.ShapeDtypeStruct((vw, DR), jnp.float32),
            jax.ShapeDtypeStruct((vq, DR), jnp.float32),
            jax.ShapeDtypeStruct((vc * vm, vs, DR), jnp.float32),
        ],
    )(weight_table, seq_table, equipment_table, core_table, metric_table,
      weight_fc_w, weight_fc_b, seq_fc_w, seq_fc_b, equipment_fc_w,
      equipment_fc_b, core_fc_w, core_fc_b, metric_fc_w, metric_fc_b)
    scm = scm3.reshape(vc * vm * vs, DR)
    ex_w = jnp.pad(exercise_table, ((0, 0), (0, DR - D)))

    mesh = plsc.VectorSubcoreMesh(core_axis_name="c", subcore_axis_name="s")
    sc = pl.kernel(
        functools.partial(_sc_body, n_chunks),
        out_type=jax.ShapeDtypeStruct((N, DO), jnp.float32),
        mesh=mesh,
        scratch_types=[
            pltpu.VMEM((SUB,), jnp.int32),
            pltpu.VMEM((SUB,), jnp.int32),
            pltpu.VMEM((SUB,), jnp.int32),
            pltpu.VMEM((SUB,), jnp.int32),
            pltpu.VMEM((SUB,), jnp.int32),
            pltpu.VMEM((SUB,), jnp.int32),
            pltpu.VMEM((SUB,), jnp.int32),
            pltpu.VMEM((CHUNK, DR), jnp.float32),
            pltpu.VMEM((CHUNK, DR), jnp.float32),
            pltpu.VMEM((CHUNK, DR), jnp.float32),
            pltpu.VMEM((CHUNK, DR), jnp.float32),
            pltpu.VMEM((CHUNK, DO), jnp.float32),
            pltpu.SemaphoreType.DMA,
        ],
    )
    out = sc(exercise_id.reshape(N),
             weight_id.reshape(N),
             equipment_id.reshape(N),
             exercise_sequence.reshape(N),
             core.reshape(N),
             metric_type.reshape(N),
             ex_w, wp, ep, scm)
    return out[:, :D].reshape(B, L, D)
